# baseline (speedup 1.0000x reference)
import functools

import jax
import jax.numpy as jnp
from jax import lax
from jax.experimental import pallas as pl
from jax.experimental.pallas import tpu as pltpu
from jax.experimental.pallas import tpu_sc as plsc

N_NODES = 10000
D = 128
N_PAD = 10240
TRASH = 10200
DW = 144
NC, NS = 2, 16
NW = NC * NS
CHUNK = 128
STEPS = 82
PAIRS = STEPS // 2
E_PAD = NW * STEPS * CHUNK
E_TOT = 320000 + N_NODES
BLK = 1280


def _mm2_body(x_ref, w_ref, t_ref):
    h = pl.program_id(0)
    m = jnp.dot(x_ref[...], w_ref[0], preferred_element_type=jnp.float32)
    col = lax.broadcasted_iota(jnp.int32, (BLK, DW), 1)
    ones = jnp.where(h == 0, 1.0, 0.0)
    t_ref[0] = jnp.where(col == D, ones, jnp.pad(m, ((0, 0), (0, DW - D))))


def _mm2(x, w2):
    n = x.shape[0]
    return pl.pallas_call(
        _mm2_body,
        grid=(2, n // BLK),
        in_specs=[pl.BlockSpec((BLK, D), lambda h, i: (i, 0)),
                  pl.BlockSpec((1, D, D), lambda h, i: (h, 0, 0))],
        out_specs=pl.BlockSpec((1, BLK, DW), lambda h, i: (h, i, 0)),
        out_shape=jax.ShapeDtypeStruct((2, n, DW), jnp.float32),
    )(x, w2)


def _mid_body(a0_ref, a1_ref, b_ref, w_ref, t_ref):
    h = pl.program_id(0)
    i = pl.program_id(1)
    v = a0_ref[...] + a1_ref[...]
    x = jnp.maximum(v[:, :D] / (v[:, D:D + 1] + 1e-16) + b_ref[...], 0.0)
    rows = i * BLK + lax.broadcasted_iota(jnp.int32, x.shape, 0)
    x = jnp.where(rows < N_NODES, x, 0.0)
    m = jnp.dot(x, w_ref[0], preferred_element_type=jnp.float32)
    col = lax.broadcasted_iota(jnp.int32, (BLK, DW), 1)
    ones = jnp.where(h == 0, 1.0, 0.0)
    t_ref[0] = jnp.where(col == D, ones, jnp.pad(m, ((0, 0), (0, DW - D))))


def _mid(a0, a1, b, w2):
    return pl.pallas_call(
        _mid_body,
        grid=(2, N_PAD // BLK),
        in_specs=[pl.BlockSpec((BLK, DW), lambda h, i: (i, 0)),
                  pl.BlockSpec((BLK, DW), lambda h, i: (i, 0)),
                  pl.BlockSpec((D,), lambda h, i: (0,)),
                  pl.BlockSpec((1, D, D), lambda h, i: (h, 0, 0))],
        out_specs=pl.BlockSpec((1, BLK, DW), lambda h, i: (h, i, 0)),
        out_shape=jax.ShapeDtypeStruct((2, N_PAD, DW), jnp.float32),
    )(a0, a1, b, w2)


def _fin_body(a0_ref, a1_ref, b_ref, o_ref):
    v = a0_ref[...] + a1_ref[...]
    o_ref[...] = jnp.maximum(v[:, :D] / (v[:, D:D + 1] + 1e-16) + b_ref[...], 0.0)


def _fin(a0, a1, b):
    blk = 1000
    return pl.pallas_call(
        _fin_body,
        grid=(N_NODES // blk,),
        in_specs=[pl.BlockSpec((blk, DW), lambda i: (i, 0)),
                  pl.BlockSpec((blk, DW), lambda i: (i, 0)),
                  pl.BlockSpec((D,), lambda i: (0,))],
        out_specs=pl.BlockSpec((blk, D), lambda i: (i, 0)),
        out_shape=jax.ShapeDtypeStruct((N_NODES, D), jnp.float32),
    )(a0, a1, b)


def _sc_edge_body(t_hbm, att_hbm, idx_hbm, out_hbm,
                  iv0, iv1, att_v, bufa, bufb, acc_sh, sga, sgb, ss, si0, si1):
    cid = lax.axis_index("c")
    sid = lax.axis_index("s")
    wid = cid * NS + sid
    iv = (iv0, iv1)
    si = (si0, si1)

    def zrow(r, c):
        for j in range(DW // 16):
            bufa[r, pl.ds(j * 16, 16)] = jnp.zeros((16,), jnp.float32)
        return c
    lax.fori_loop(0, CHUNK, zrow, 0)
    rows_per_tile = N_PAD // NS
    for k in range(rows_per_tile // CHUNK):
        pltpu.sync_copy(bufa,
                        acc_sh.at[pl.ds(sid * rows_per_tile + k * CHUNK, CHUNK)])

    pltpu.sync_copy(att_hbm, att_v)
    att_c = [att_v[pl.ds(j * 16, 16)] for j in range(8)]
    e0 = jnp.where(lax.iota(jnp.int32, 16) == 0, 1.0, 0.0)
    plsc.subcore_barrier()

    def scat_start(q):
        pltpu.async_copy(bufa, acc_sh.at[iv[q].at[2]], ss, add=True)

    def scat_wait(q):
        pltpu.make_async_copy(bufa, acc_sh.at[iv[q].at[2]], ss).wait()

    def idx_start(q, g):
        pltpu.async_copy(idx_hbm.at[wid, g], iv[q], si[q])

    def idx_wait(q, g):
        pltpu.make_async_copy(idx_hbm.at[wid, g], iv[q], si[q]).wait()

    def compute():
        @plsc.parallel_loop(0, CHUNK, 1, unroll=4)
        def edge(e):
            a = [bufa[e, pl.ds(j * 16, 16)] for j in range(8)]
            p = []
            for j in range(8):
                s = a[j] + bufb[e, pl.ds(j * 16, 16)]
                p.append(att_c[j] * jnp.maximum(s, 0.2 * s))
            q = [p[0] + p[1], p[2] + p[3], p[4] + p[5], p[6] + p[7]]
            acc = (q[0] + q[1]) + (q[2] + q[3])
            ex = jnp.exp(lax.broadcast(jnp.sum(acc), (16,)))
            for j in range(8):
                bufa[e, pl.ds(j * 16, 16)] = ex * a[j]
            bufa[e, pl.ds(D, 16)] = ex * e0

    pltpu.sync_copy(idx_hbm.at[wid, 0], iv[0])

    def pair(s, c):
        for q in (0, 1):
            if q == 0:
                @pl.when(s > 0)
                def _():
                    idx_wait(0, 2 * s)
            else:
                idx_wait(1, 2 * s + 1)
            cpb = pltpu.async_copy(t_hbm.at[iv[q].at[1]], bufb, sgb)
            if q == 0:
                @pl.when(s > 0)
                def _():
                    scat_wait(1)
            else:
                scat_wait(0)
            cpa = pltpu.async_copy(t_hbm.at[iv[q].at[0]], bufa, sga)
            if q == 0:
                idx_start(1, 2 * s + 1)
            else:
                @pl.when(s < PAIRS - 1)
                def _():
                    idx_start(0, 2 * s + 2)
            cpb.wait()
            cpa.wait()
            compute()
            scat_start(q)
        return c
    lax.fori_loop(0, PAIRS, pair, 0)
    scat_wait(1)

    plsc.subcore_barrier()
    pltpu.sync_copy(acc_sh.at[pl.ds(sid * rows_per_tile, rows_per_tile)],
                    out_hbm.at[cid, sid])


@functools.cache
def _make_sc_edge():
    mesh = plsc.VectorSubcoreMesh(core_axis_name="c", subcore_axis_name="s")
    return pl.kernel(
        _sc_edge_body,
        out_type=jax.ShapeDtypeStruct((NC, NS, N_PAD // NS, DW), jnp.float32),
        mesh=mesh,
        scratch_types=[
            pltpu.VMEM((3, CHUNK), jnp.int32),
            pltpu.VMEM((3, CHUNK), jnp.int32),
            pltpu.VMEM((D,), jnp.float32),
            pltpu.VMEM((CHUNK, DW), jnp.float32),
            pltpu.VMEM((CHUNK, DW), jnp.float32),
            pltpu.VMEM_SHARED((N_PAD, DW), jnp.float32),
            pltpu.SemaphoreType.DMA,
            pltpu.SemaphoreType.DMA,
            pltpu.SemaphoreType.DMA,
            pltpu.SemaphoreType.DMA,
            pltpu.SemaphoreType.DMA,
        ],
        compiler_params=pltpu.CompilerParams(use_tc_tiling_on_sc=False,
                                             needs_layout_passes=False),
    )


def _sc_edge(t, att, idx):
    acc = _make_sc_edge()(t, att, idx)
    return jnp.reshape(acc, (NC, N_PAD, DW))


def kernel(node_features, Wl1, Wr1, att1, b1, Wl2, Wr2, att2, b2, edge_index):
    x0 = jnp.pad(node_features, ((0, N_PAD - N_NODES), (0, 0)))
    ei = edge_index.astype(jnp.int32)
    loop = jnp.arange(N_NODES, dtype=jnp.int32)
    pad = jnp.full((E_PAD - E_TOT,), TRASH, jnp.int32)
    src = jnp.concatenate([ei[0], loop, pad]).reshape(NW, STEPS, 1, CHUNK)
    dst = jnp.concatenate([ei[1], loop, pad]).reshape(NW, STEPS, 1, CHUNK)
    idx = jnp.concatenate([src, dst + N_PAD, dst], axis=2)

    t1 = _mm2(x0, jnp.stack([Wl1, Wr1]))
    acc1 = _sc_edge(jnp.reshape(t1, (2 * N_PAD, DW)), att1, idx)
    t2 = _mid(acc1[0], acc1[1], b1, jnp.stack([Wl2, Wr2]))
    acc2 = _sc_edge(jnp.reshape(t2, (2 * N_PAD, DW)), att2, idx)
    return _fin(acc2[0], acc2[1], b2)


# speedup vs baseline: 1.4903x; 1.4903x over previous
import functools

import jax
import jax.numpy as jnp
from jax import lax
from jax.experimental import pallas as pl
from jax.experimental.pallas import tpu as pltpu
from jax.experimental.pallas import tpu_sc as plsc

N_NODES = 10000
D = 128
N_PAD = 10240
TRASH = 10200
DW = 144
NC, NS = 2, 16
NW = NC * NS
CHUNK = 128
STEPS = 81
E_PAD = NW * STEPS * CHUNK
E_TOT = 320000 + N_NODES
BLK = 1280


def _mm2_body(x_ref, wl_ref, wr_ref, xl_ref, xr_ref):
    x = x_ref[...]
    ml = jnp.dot(x, wl_ref[...], preferred_element_type=jnp.float32)
    col = lax.broadcasted_iota(jnp.int32, (BLK, DW), 1)
    xl_ref[...] = jnp.where(col == D, 1.0, jnp.pad(ml, ((0, 0), (0, DW - D))))
    xr_ref[...] = jnp.dot(x, wr_ref[...], preferred_element_type=jnp.float32)


def _mm2(x, wl, wr):
    n = x.shape[0]
    return pl.pallas_call(
        _mm2_body,
        grid=(n // BLK,),
        in_specs=[pl.BlockSpec((BLK, D), lambda i: (i, 0)),
                  pl.BlockSpec((D, D), lambda i: (0, 0)),
                  pl.BlockSpec((D, D), lambda i: (0, 0))],
        out_specs=(pl.BlockSpec((BLK, DW), lambda i: (i, 0)),
                   pl.BlockSpec((BLK, D), lambda i: (i, 0))),
        out_shape=(jax.ShapeDtypeStruct((n, DW), jnp.float32),
                   jax.ShapeDtypeStruct((n, D), jnp.float32)),
    )(x, wl, wr)


def _mid_body(a0_ref, a1_ref, b_ref, wl_ref, wr_ref, xl_ref, xr_ref):
    i = pl.program_id(0)
    v = a0_ref[...] + a1_ref[...]
    num = v[:, :D]
    den = v[:, D:D + 1]
    x = jnp.maximum(num / (den + 1e-16) + b_ref[...], 0.0)
    rows = i * BLK + lax.broadcasted_iota(jnp.int32, x.shape, 0)
    x = jnp.where(rows < N_NODES, x, 0.0)
    ml = jnp.dot(x, wl_ref[...], preferred_element_type=jnp.float32)
    col = lax.broadcasted_iota(jnp.int32, (BLK, DW), 1)
    xl_ref[...] = jnp.where(col == D, 1.0, jnp.pad(ml, ((0, 0), (0, DW - D))))
    xr_ref[...] = jnp.dot(x, wr_ref[...], preferred_element_type=jnp.float32)


def _mid(a0, a1, b, wl, wr):
    return pl.pallas_call(
        _mid_body,
        grid=(N_PAD // BLK,),
        in_specs=[pl.BlockSpec((BLK, DW), lambda i: (i, 0)),
                  pl.BlockSpec((BLK, DW), lambda i: (i, 0)),
                  pl.BlockSpec((D,), lambda i: (0,)),
                  pl.BlockSpec((D, D), lambda i: (0, 0)),
                  pl.BlockSpec((D, D), lambda i: (0, 0))],
        out_specs=(pl.BlockSpec((BLK, DW), lambda i: (i, 0)),
                   pl.BlockSpec((BLK, D), lambda i: (i, 0))),
        out_shape=(jax.ShapeDtypeStruct((N_PAD, DW), jnp.float32),
                   jax.ShapeDtypeStruct((N_PAD, D), jnp.float32)),
    )(a0, a1, b, wl, wr)


def _fin_body(a0_ref, a1_ref, b_ref, o_ref):
    v = a0_ref[...] + a1_ref[...]
    o_ref[...] = jnp.maximum(v[:, :D] / (v[:, D:D + 1] + 1e-16) + b_ref[...], 0.0)


def _fin(a0, a1, b):
    blk = 1000
    return pl.pallas_call(
        _fin_body,
        grid=(N_NODES // blk,),
        in_specs=[pl.BlockSpec((blk, DW), lambda i: (i, 0)),
                  pl.BlockSpec((blk, DW), lambda i: (i, 0)),
                  pl.BlockSpec((D,), lambda i: (0,))],
        out_specs=pl.BlockSpec((blk, D), lambda i: (i, 0)),
        out_shape=jax.ShapeDtypeStruct((N_NODES, D), jnp.float32),
    )(a0, a1, b)


def _sc_edge_body(xl_hbm, xr_hbm, att_hbm, idx_hbm, out_hbm,
                  iv, att_v, bufa, bufb, acc_sh, sema, semb):
    cid = lax.axis_index("c")
    sid = lax.axis_index("s")
    wid = cid * NS + sid

    def zrow(r, c):
        for j in range(DW // 16):
            bufa[r, pl.ds(j * 16, 16)] = jnp.zeros((16,), jnp.float32)
        return c
    lax.fori_loop(0, CHUNK, zrow, 0)
    rows_per_tile = N_PAD // NS
    for k in range(rows_per_tile // CHUNK):
        pltpu.sync_copy(bufa, acc_sh.at[pl.ds(sid * rows_per_tile + k * CHUNK, CHUNK)])

    pltpu.sync_copy(att_hbm, att_v)
    att_c = [att_v[pl.ds(j * 16, 16)] for j in range(8)]
    e0 = jnp.where(lax.iota(jnp.int32, 16) == 0, 1.0, 0.0)
    plsc.subcore_barrier()

    def step(g, c):
        pltpu.sync_copy(idx_hbm.at[wid, g], iv)
        cpa = pltpu.async_copy(xl_hbm.at[iv.at[0]], bufa, sema)
        cpb = pltpu.async_copy(xr_hbm.at[iv.at[1]], bufb, semb)
        cpa.wait()
        cpb.wait()

        @plsc.parallel_loop(0, CHUNK, 1, unroll=4)
        def edge(e):
            a = [bufa[e, pl.ds(j * 16, 16)] for j in range(8)]
            p = []
            for j in range(8):
                s = a[j] + bufb[e, pl.ds(j * 16, 16)]
                p.append(att_c[j] * jnp.maximum(s, 0.2 * s))
            q = [p[0] + p[1], p[2] + p[3], p[4] + p[5], p[6] + p[7]]
            acc = (q[0] + q[1]) + (q[2] + q[3])
            ex = jnp.exp(lax.broadcast(jnp.sum(acc), (16,)))
            for j in range(8):
                bufa[e, pl.ds(j * 16, 16)] = ex * a[j]
            bufa[e, pl.ds(D, 16)] = ex * e0

        pltpu.sync_copy(bufa, acc_sh.at[iv.at[1]], add=True)
        return c
    lax.fori_loop(0, STEPS, step, 0)

    plsc.subcore_barrier()
    pltpu.sync_copy(acc_sh.at[pl.ds(sid * rows_per_tile, rows_per_tile)],
                    out_hbm.at[cid, sid])


@functools.cache
def _make_sc_edge():
    mesh = plsc.VectorSubcoreMesh(core_axis_name="c", subcore_axis_name="s")
    return pl.kernel(
        _sc_edge_body,
        out_type=jax.ShapeDtypeStruct((NC, NS, N_PAD // NS, DW), jnp.float32),
        mesh=mesh,
        scratch_types=[
            pltpu.VMEM((2, CHUNK), jnp.int32),
            pltpu.VMEM((D,), jnp.float32),
            pltpu.VMEM((CHUNK, DW), jnp.float32),
            pltpu.VMEM((CHUNK, D), jnp.float32),
            pltpu.VMEM_SHARED((N_PAD, DW), jnp.float32),
            pltpu.SemaphoreType.DMA,
            pltpu.SemaphoreType.DMA,
        ],
        compiler_params=pltpu.CompilerParams(use_tc_tiling_on_sc=False,
                                             needs_layout_passes=False),
    )


def _sc_edge(xl, xr, att, idx):
    acc = _make_sc_edge()(xl, xr, att, idx)
    return jnp.reshape(acc, (NC, N_PAD, DW))


def kernel(node_features, Wl1, Wr1, att1, b1, Wl2, Wr2, att2, b2, edge_index):
    x0 = jnp.pad(node_features, ((0, N_PAD - N_NODES), (0, 0)))
    ei = edge_index.astype(jnp.int32)
    loop = jnp.arange(N_NODES, dtype=jnp.int32)
    pad = jnp.full((E_PAD - E_TOT,), TRASH, jnp.int32)
    src = jnp.concatenate([ei[0], loop, pad]).reshape(NW, STEPS, 1, CHUNK)
    dst = jnp.concatenate([ei[1], loop, pad]).reshape(NW, STEPS, 1, CHUNK)
    idx = jnp.concatenate([src, dst], axis=2)

    xl1, xr1 = _mm2(x0, Wl1, Wr1)
    acc1 = _sc_edge(xl1, xr1, att1, idx)
    xl2, xr2 = _mid(acc1[0], acc1[1], b1, Wl2, Wr2)
    acc2 = _sc_edge(xl2, xr2, att2, idx)
    return _fin(acc2[0], acc2[1], b2)
